# baseline (speedup 1.0000x reference)
import jax
import jax.numpy as jnp
from jax import lax
from jax.experimental import pallas as pl
from jax.experimental.pallas import tpu as pltpu
from jax.experimental.pallas import tpu_sc as plsc

NC = 2
NS = 16
L = 16
NW = NC * NS

BATCH = 4
SEQ = 2048
D = 1024

S_SC = 256
S_PER_W = S_SC // NW
CH_ROWS = 8
K_CH = S_PER_W // CH_ROWS
NIT = BATCH * K_CH
DEPTH = 7
AHEAD = 5

BLOCK_S = 256
MG_BS = 64
TC_BLOCKS = (SEQ - S_SC) // BLOCK_S
MG_BLOCKS = S_SC // MG_BS


def _sc_body(x_hbm, pe_hbm, out_hbm, peb, *rest):
    bufs = list(rest[:DEPTH])
    pes = rest[DEPTH]
    isems = list(rest[DEPTH + 1:DEPTH + 1 + DEPTH])
    osems = list(rest[DEPTH + 1 + DEPTH:DEPTH + 1 + 2 * DEPTH])

    w = lax.axis_index("s") * NC + lax.axis_index("c")
    s0 = w * S_PER_W

    pe_h = []
    for p in range(K_CH):
        pe_h.append(pltpu.async_copy(
            pe_hbm.at[pl.ds(s0 + p * CH_ROWS, CH_ROWS)],
            peb.at[pl.ds(p * CH_ROWS, CH_ROWS)], pes))

    def chunk_of(t):
        b, k = divmod(t, K_CH)
        return b, k

    def start_in(t):
        b, k = chunk_of(t)
        return pltpu.async_copy(
            x_hbm.at[b, pl.ds(s0 + k * CH_ROWS, CH_ROWS)],
            bufs[t % DEPTH], isems[t % DEPTH])

    def start_out(t):
        b, k = chunk_of(t)
        return pltpu.async_copy(
            bufs[t % DEPTH],
            out_hbm.at[b, pl.ds(s0 + k * CH_ROWS, CH_ROWS)], osems[t % DEPTH])

    in_h = {}
    out_h = {}
    for t in range(AHEAD):
        in_h[t] = start_in(t)

    pe_waited = 0
    for t in range(NIT):
        b, k = chunk_of(t)
        xb = bufs[t % DEPTH]
        while pe_waited < min(k + 1, K_CH):
            pe_h[pe_waited].wait()
            pe_waited += 1
        in_h[t].wait()

        def row_body(r, carry, _xb=xb, _k=k):
            pr = _k * CH_ROWS + r

            @plsc.parallel_loop(0, D, step=L, unroll=8)
            def _add(i):
                _xb[r, pl.ds(i, L)] = _xb[r, pl.ds(i, L)] + peb[pr, pl.ds(i, L)]

            return carry

        lax.fori_loop(0, CH_ROWS, row_body, 0)

        out_h[t] = start_out(t)
        nt = t + AHEAD
        if nt < NIT:
            prev = nt - DEPTH
            if prev >= 0:
                out_h[prev].wait()
            in_h[nt] = start_in(nt)

    for t in range(max(0, NIT - DEPTH), NIT):
        if t in out_h:
            out_h[t].wait()


def _sc_call(x, pe):
    mesh = plsc.VectorSubcoreMesh(core_axis_name="c", subcore_axis_name="s")
    return pl.kernel(
        _sc_body,
        out_type=jax.ShapeDtypeStruct((BATCH, S_SC, D), jnp.float32),
        mesh=mesh,
        scratch_types=(
            [pltpu.VMEM((S_PER_W, D), jnp.float32)]
            + [pltpu.VMEM((CH_ROWS, D), jnp.float32) for _ in range(DEPTH)]
            + [pltpu.SemaphoreType.DMA for _ in range(1 + 2 * DEPTH)]
        ),
    )(x, pe)


def _tc_add_body(x_ref, pe_ref, o_ref):
    o_ref[...] = x_ref[...] + pe_ref[...][None, :, :]


def _tc_add(x, pe):
    base = S_SC // BLOCK_S
    return pl.pallas_call(
        _tc_add_body,
        grid=(TC_BLOCKS,),
        in_specs=[
            pl.BlockSpec((BATCH, BLOCK_S, D), lambda i: (0, i + base, 0)),
            pl.BlockSpec((BLOCK_S, D), lambda i: (i + base, 0)),
        ],
        out_specs=pl.BlockSpec((BATCH, BLOCK_S, D), lambda i: (0, i + base, 0)),
        out_shape=jax.ShapeDtypeStruct((BATCH, SEQ, D), jnp.float32),
    )(x, pe)


def _merge_body(osc_ref, _full_ref, o_ref):
    o_ref[...] = osc_ref[...]


def _merge(o_sc, o_full):
    return pl.pallas_call(
        _merge_body,
        grid=(MG_BLOCKS,),
        in_specs=[
            pl.BlockSpec((BATCH, MG_BS, D), lambda i: (0, i, 0)),
            pl.BlockSpec(memory_space=pltpu.MemorySpace.HBM),
        ],
        out_specs=pl.BlockSpec((BATCH, MG_BS, D), lambda i: (0, i, 0)),
        out_shape=jax.ShapeDtypeStruct((BATCH, SEQ, D), jnp.float32),
        input_output_aliases={1: 0},
    )(o_sc, o_full)


def kernel(x, pos_table):
    batch, seq_len, d_model = x.shape
    pe = pos_table[:seq_len]
    o_sc = _sc_call(x, pe)
    o_full = _tc_add(x, pe)
    return _merge(o_sc, o_full)


# speedup vs baseline: 1.0131x; 1.0131x over previous
import jax
import jax.numpy as jnp
from jax import lax
from jax.experimental import pallas as pl
from jax.experimental.pallas import tpu as pltpu
from jax.experimental.pallas import tpu_sc as plsc

NC = 2
NS = 16
L = 16
NW = NC * NS

BATCH = 4
SEQ = 2048
D = 1024

S_SC = 256
S_PER_W = S_SC // NW
CH_ROWS = 8
K_CH = S_PER_W // CH_ROWS
NIT = BATCH * K_CH
DEPTH = 7
AHEAD = 5

BLOCK_S = 256
MG_BS = 256
TC_BLOCKS = (SEQ - S_SC) // BLOCK_S
MG_BLOCKS = S_SC // MG_BS


def _sc_body(x_hbm, pe_hbm, out_hbm, peb, *rest):
    bufs = list(rest[:DEPTH])
    pes = rest[DEPTH]
    isems = list(rest[DEPTH + 1:DEPTH + 1 + DEPTH])
    osems = list(rest[DEPTH + 1 + DEPTH:DEPTH + 1 + 2 * DEPTH])

    w = lax.axis_index("s") * NC + lax.axis_index("c")
    s0 = w * S_PER_W

    pe_h = []
    for p in range(K_CH):
        pe_h.append(pltpu.async_copy(
            pe_hbm.at[pl.ds(s0 + p * CH_ROWS, CH_ROWS)],
            peb.at[pl.ds(p * CH_ROWS, CH_ROWS)], pes))

    def chunk_of(t):
        b, k = divmod(t, K_CH)
        return b, k

    def start_in(t):
        b, k = chunk_of(t)
        return pltpu.async_copy(
            x_hbm.at[b, pl.ds(s0 + k * CH_ROWS, CH_ROWS)],
            bufs[t % DEPTH], isems[t % DEPTH])

    def start_out(t):
        b, k = chunk_of(t)
        return pltpu.async_copy(
            bufs[t % DEPTH],
            out_hbm.at[b, pl.ds(s0 + k * CH_ROWS, CH_ROWS)], osems[t % DEPTH])

    in_h = {}
    out_h = {}
    for t in range(AHEAD):
        in_h[t] = start_in(t)

    pe_waited = 0
    for t in range(NIT):
        b, k = chunk_of(t)
        xb = bufs[t % DEPTH]
        while pe_waited < min(k + 1, K_CH):
            pe_h[pe_waited].wait()
            pe_waited += 1
        in_h[t].wait()

        def row_body(r, carry, _xb=xb, _k=k):
            pr = _k * CH_ROWS + r

            @plsc.parallel_loop(0, D, step=L, unroll=8)
            def _add(i):
                _xb[r, pl.ds(i, L)] = _xb[r, pl.ds(i, L)] + peb[pr, pl.ds(i, L)]

            return carry

        lax.fori_loop(0, CH_ROWS, row_body, 0)

        out_h[t] = start_out(t)
        nt = t + AHEAD
        if nt < NIT:
            prev = nt - DEPTH
            if prev >= 0:
                out_h[prev].wait()
            in_h[nt] = start_in(nt)

    for t in range(max(0, NIT - DEPTH), NIT):
        if t in out_h:
            out_h[t].wait()


def _sc_call(x, pe):
    mesh = plsc.VectorSubcoreMesh(core_axis_name="c", subcore_axis_name="s")
    return pl.kernel(
        _sc_body,
        out_type=jax.ShapeDtypeStruct((BATCH, S_SC, D), jnp.float32),
        mesh=mesh,
        scratch_types=(
            [pltpu.VMEM((S_PER_W, D), jnp.float32)]
            + [pltpu.VMEM((CH_ROWS, D), jnp.float32) for _ in range(DEPTH)]
            + [pltpu.SemaphoreType.DMA for _ in range(1 + 2 * DEPTH)]
        ),
    )(x, pe)


def _tc_add_body(x_ref, pe_ref, o_ref):
    o_ref[...] = x_ref[...] + pe_ref[...][None, :, :]


def _tc_add(x, pe):
    base = S_SC // BLOCK_S
    return pl.pallas_call(
        _tc_add_body,
        grid=(TC_BLOCKS,),
        in_specs=[
            pl.BlockSpec((BATCH, BLOCK_S, D), lambda i: (0, i + base, 0)),
            pl.BlockSpec((BLOCK_S, D), lambda i: (i + base, 0)),
        ],
        out_specs=pl.BlockSpec((BATCH, BLOCK_S, D), lambda i: (0, i + base, 0)),
        out_shape=jax.ShapeDtypeStruct((BATCH, SEQ, D), jnp.float32),
    )(x, pe)


def _merge_body(osc_ref, _full_ref, o_ref):
    o_ref[...] = osc_ref[...]


def _merge(o_sc, o_full):
    return pl.pallas_call(
        _merge_body,
        grid=(MG_BLOCKS,),
        in_specs=[
            pl.BlockSpec((BATCH, MG_BS, D), lambda i: (0, i, 0)),
            pl.BlockSpec(memory_space=pltpu.MemorySpace.HBM),
        ],
        out_specs=pl.BlockSpec((BATCH, MG_BS, D), lambda i: (0, i, 0)),
        out_shape=jax.ShapeDtypeStruct((BATCH, SEQ, D), jnp.float32),
        input_output_aliases={1: 0},
    )(o_sc, o_full)


def kernel(x, pos_table):
    batch, seq_len, d_model = x.shape
    pe = pos_table[:seq_len]
    o_sc = _sc_call(x, pe)
    o_full = _tc_add(x, pe)
    return _merge(o_sc, o_full)


# speedup vs baseline: 1.0235x; 1.0103x over previous
import jax
import jax.numpy as jnp
from jax import lax
from jax.experimental import pallas as pl
from jax.experimental.pallas import tpu as pltpu
from jax.experimental.pallas import tpu_sc as plsc

NC = 2
NS = 16
L = 16
NW = NC * NS

BATCH = 4
SEQ = 2048
D = 1024

S_SC = 256
S_PER_W = S_SC // NW
CH_ROWS = 8
K_CH = S_PER_W // CH_ROWS
NIT = BATCH * K_CH
DEPTH = 7
AHEAD = 5

BLOCK_S = 256
MG_BS = 256
TC_BLOCKS = (SEQ - S_SC) // BLOCK_S
MG_BLOCKS = S_SC // MG_BS


def _sc_body(x_hbm, pe_hbm, out_hbm, peb, *rest):
    bufs = list(rest[:DEPTH])
    pes = rest[DEPTH]
    isems = list(rest[DEPTH + 1:DEPTH + 1 + DEPTH])
    osems = list(rest[DEPTH + 1 + DEPTH:DEPTH + 1 + 2 * DEPTH])

    w = lax.axis_index("s") * NC + lax.axis_index("c")
    s0 = w * S_PER_W

    pe_h = []
    for p in range(K_CH):
        pe_h.append(pltpu.async_copy(
            pe_hbm.at[pl.ds(s0 + p * CH_ROWS, CH_ROWS)],
            peb.at[pl.ds(p * CH_ROWS, CH_ROWS)], pes))

    def chunk_of(t):
        b, k = divmod(t, K_CH)
        return b, k

    def start_in(t):
        b, k = chunk_of(t)
        return pltpu.async_copy(
            x_hbm.at[b, pl.ds(s0 + k * CH_ROWS, CH_ROWS)],
            bufs[t % DEPTH], isems[t % DEPTH])

    def start_out(t):
        b, k = chunk_of(t)
        return pltpu.async_copy(
            bufs[t % DEPTH],
            out_hbm.at[b, pl.ds(s0 + k * CH_ROWS, CH_ROWS)], osems[t % DEPTH])

    in_h = {}
    out_h = {}
    for t in range(min(AHEAD, NIT)):
        in_h[t] = start_in(t)

    pe_waited = 0
    for t in range(NIT):
        b, k = chunk_of(t)
        xb = bufs[t % DEPTH]
        while pe_waited < min(k + 1, K_CH):
            pe_h[pe_waited].wait()
            pe_waited += 1
        in_h[t].wait()

        def row_body(r, carry, _xb=xb, _k=k):
            pr = _k * CH_ROWS + r

            @plsc.parallel_loop(0, D, step=L, unroll=8)
            def _add(i):
                _xb[r, pl.ds(i, L)] = _xb[r, pl.ds(i, L)] + peb[pr, pl.ds(i, L)]

            return carry

        lax.fori_loop(0, CH_ROWS, row_body, 0)

        out_h[t] = start_out(t)
        nt = t + AHEAD
        if nt < NIT:
            prev = nt - DEPTH
            if prev >= 0:
                out_h[prev].wait()
            in_h[nt] = start_in(nt)

    for t in range(max(0, NIT - DEPTH), NIT):
        if t in out_h:
            out_h[t].wait()


def _sc_call(x, pe):
    mesh = plsc.VectorSubcoreMesh(core_axis_name="c", subcore_axis_name="s")
    return pl.kernel(
        _sc_body,
        out_type=jax.ShapeDtypeStruct((BATCH, S_SC, D), jnp.float32),
        mesh=mesh,
        scratch_types=(
            [pltpu.VMEM((S_PER_W, D), jnp.float32)]
            + [pltpu.VMEM((CH_ROWS, D), jnp.float32) for _ in range(DEPTH)]
            + [pltpu.SemaphoreType.DMA for _ in range(1 + 2 * DEPTH)]
        ),
    )(x, pe)


def _tc_add_body(x_ref, pe_ref, o_ref):
    o_ref[...] = x_ref[...] + pe_ref[...][None, :, :]


def _tc_add(x, pe):
    base = S_SC // BLOCK_S
    return pl.pallas_call(
        _tc_add_body,
        grid=(TC_BLOCKS,),
        in_specs=[
            pl.BlockSpec((BATCH, BLOCK_S, D), lambda i: (0, i + base, 0)),
            pl.BlockSpec((BLOCK_S, D), lambda i: (i + base, 0)),
        ],
        out_specs=pl.BlockSpec((BATCH, BLOCK_S, D), lambda i: (0, i + base, 0)),
        out_shape=jax.ShapeDtypeStruct((BATCH, SEQ, D), jnp.float32),
    )(x, pe)


def _merge_body(osc_ref, _full_ref, o_ref):
    o_ref[...] = osc_ref[...]


def _merge(o_sc, o_full):
    return pl.pallas_call(
        _merge_body,
        grid=(MG_BLOCKS,),
        in_specs=[
            pl.BlockSpec((BATCH, MG_BS, D), lambda i: (0, i, 0)),
            pl.BlockSpec(memory_space=pltpu.MemorySpace.HBM),
        ],
        out_specs=pl.BlockSpec((BATCH, MG_BS, D), lambda i: (0, i, 0)),
        out_shape=jax.ShapeDtypeStruct((BATCH, SEQ, D), jnp.float32),
        input_output_aliases={1: 0},
    )(o_sc, o_full)


def kernel(x, pos_table):
    batch, seq_len, d_model = x.shape
    pe = pos_table[:seq_len]
    o_sc = _sc_call(x, pe)
    o_full = _tc_add(x, pe)
    return _merge(o_sc, o_full)
